# baseline (speedup 1.0000x reference)
import jax
import jax.numpy as jnp
from jax.experimental import pallas as pl
from jax.experimental.pallas import tpu as pltpu

NZ = 34
E = 7
S = 40
KX = E * S
R = 32


def _moe_kernel(eta_ref, xq_ref, xs_ref, wq_ref, ws_ref, out_ref):
    _, ny, nx = xq_ref.shape
    n = ny * nx
    eta = eta_ref[...].reshape(1, n)
    xq = xq_ref[...].astype(jnp.bfloat16).reshape(NZ, n)
    xs = xs_ref[...].astype(jnp.bfloat16).reshape(NZ, n)
    pad = jnp.zeros((S - NZ - 1, n), jnp.bfloat16)
    one = jnp.ones((1, n), jnp.bfloat16)
    xaugq = jnp.concatenate([xq, one, pad], axis=0)
    xaugs = jnp.concatenate([xs, one, pad], axis=0)
    zed = jnp.zeros((S, n), jnp.bfloat16)
    xkq = jnp.concatenate([jnp.where(eta == e, xaugq, zed) for e in range(E)], axis=0)
    xks = jnp.concatenate([jnp.where(eta == e, xaugs, zed) for e in range(E)], axis=0)
    oq = jnp.dot(wq_ref[...], xkq, preferred_element_type=jnp.float32)
    osli = jnp.dot(ws_ref[...], xks, preferred_element_type=jnp.float32)
    out_ref[0] = oq.reshape(NZ, ny, nx)
    out_ref[1] = osli.reshape(NZ, ny, nx)


def _stack_weights(W, b):
    pad = jnp.zeros((E, S - NZ - 1, NZ), W.dtype)
    wt = jnp.concatenate([jnp.swapaxes(W, 1, 2), b[:, None, :], pad], axis=1)
    return wt.reshape(KX, NZ).T.astype(jnp.bfloat16)


def kernel(x_QT, x_SLI, eta, W_QT, b_QT, W_SLI, b_SLI):
    NY, NX = eta.shape
    G = NY // R
    wq = _stack_weights(W_QT, b_QT)
    ws = _stack_weights(W_SLI, b_SLI)
    return pl.pallas_call(
        _moe_kernel,
        grid=(G,),
        in_specs=[
            pl.BlockSpec((R, NX), lambda i: (i, 0)),
            pl.BlockSpec((NZ, R, NX), lambda i: (0, i, 0)),
            pl.BlockSpec((NZ, R, NX), lambda i: (0, i, 0)),
            pl.BlockSpec((NZ, KX), lambda i: (0, 0)),
            pl.BlockSpec((NZ, KX), lambda i: (0, 0)),
        ],
        out_specs=pl.BlockSpec((2, NZ, R, NX), lambda i: (0, 0, i, 0)),
        out_shape=jax.ShapeDtypeStruct((2, NZ, NY, NX), jnp.float32),
        compiler_params=pltpu.CompilerParams(
            dimension_semantics=("parallel",), vmem_limit_bytes=100*1024*1024),
    )(eta, x_QT, x_SLI, wq, ws)


# speedup vs baseline: 1.0100x; 1.0100x over previous
import jax
import jax.numpy as jnp
from jax.experimental import pallas as pl
from jax.experimental.pallas import tpu as pltpu

NZ = 34
E = 7
S = 40
KX = E * S
R = 16


def _moe_kernel(eta_ref, xq_ref, xs_ref, wq_ref, ws_ref, out_ref):
    _, ny, nx = xq_ref.shape
    n = ny * nx
    eta = eta_ref[...].reshape(1, n)
    xq = xq_ref[...].astype(jnp.bfloat16).reshape(NZ, n)
    xs = xs_ref[...].astype(jnp.bfloat16).reshape(NZ, n)
    pad = jnp.zeros((S - NZ - 1, n), jnp.bfloat16)
    one = jnp.ones((1, n), jnp.bfloat16)
    xaugq = jnp.concatenate([xq, one, pad], axis=0)
    xaugs = jnp.concatenate([xs, one, pad], axis=0)
    zed = jnp.zeros((S, n), jnp.bfloat16)
    xkq = jnp.concatenate([jnp.where(eta == e, xaugq, zed) for e in range(E)], axis=0)
    xks = jnp.concatenate([jnp.where(eta == e, xaugs, zed) for e in range(E)], axis=0)
    oq = jnp.dot(wq_ref[...], xkq, preferred_element_type=jnp.float32)
    osli = jnp.dot(ws_ref[...], xks, preferred_element_type=jnp.float32)
    out_ref[0] = oq.reshape(NZ, ny, nx)
    out_ref[1] = osli.reshape(NZ, ny, nx)


def _stack_weights(W, b):
    pad = jnp.zeros((E, S - NZ - 1, NZ), W.dtype)
    wt = jnp.concatenate([jnp.swapaxes(W, 1, 2), b[:, None, :], pad], axis=1)
    return wt.reshape(KX, NZ).T.astype(jnp.bfloat16)


def kernel(x_QT, x_SLI, eta, W_QT, b_QT, W_SLI, b_SLI):
    NY, NX = eta.shape
    G = NY // R
    wq = _stack_weights(W_QT, b_QT)
    ws = _stack_weights(W_SLI, b_SLI)
    return pl.pallas_call(
        _moe_kernel,
        grid=(G,),
        in_specs=[
            pl.BlockSpec((R, NX), lambda i: (i, 0)),
            pl.BlockSpec((NZ, R, NX), lambda i: (0, i, 0)),
            pl.BlockSpec((NZ, R, NX), lambda i: (0, i, 0)),
            pl.BlockSpec((NZ, KX), lambda i: (0, 0)),
            pl.BlockSpec((NZ, KX), lambda i: (0, 0)),
        ],
        out_specs=pl.BlockSpec((2, NZ, R, NX), lambda i: (0, 0, i, 0)),
        out_shape=jax.ShapeDtypeStruct((2, NZ, NY, NX), jnp.float32),
        compiler_params=pltpu.CompilerParams(
            dimension_semantics=("parallel",), vmem_limit_bytes=100*1024*1024),
    )(eta, x_QT, x_SLI, wq, ws)


# speedup vs baseline: 1.0388x; 1.0285x over previous
import jax
import jax.numpy as jnp
from jax.experimental import pallas as pl
from jax.experimental.pallas import tpu as pltpu

NZ = 34
E = 7
S = 40
KX = E * S
R = 16


def _moe_kernel(eta_ref, xq_ref, xs_ref, w_ref, out_ref):
    _, ny, nx = xq_ref.shape
    n = ny * nx
    eta = eta_ref[...].reshape(1, n)
    xq = xq_ref[...].astype(jnp.bfloat16).reshape(NZ, n)
    xs = xs_ref[...].astype(jnp.bfloat16).reshape(NZ, n)
    pad = jnp.zeros((S - NZ - 1, n), jnp.bfloat16)
    one = jnp.ones((1, n), jnp.bfloat16)
    xaugq = jnp.concatenate([xq, one, pad], axis=0)
    xaugs = jnp.concatenate([xs, one, pad], axis=0)
    zed = jnp.zeros((S, n), jnp.bfloat16)
    xkq = jnp.concatenate([jnp.where(eta == e, xaugq, zed) for e in range(E)], axis=0)
    xks = jnp.concatenate([jnp.where(eta == e, xaugs, zed) for e in range(E)], axis=0)
    oq = jnp.dot(w_ref[0], xkq, preferred_element_type=jnp.float32)
    osli = jnp.dot(w_ref[1], xks, preferred_element_type=jnp.float32)
    out_ref[0] = oq.reshape(NZ, ny, nx)
    out_ref[1] = osli.reshape(NZ, ny, nx)


def _stack_weights(W, b):
    pad = jnp.zeros((2, E, S - NZ - 1, NZ), W.dtype)
    wt = jnp.concatenate([jnp.swapaxes(W, 2, 3), b[:, :, None, :], pad], axis=2)
    return jnp.swapaxes(wt.reshape(2, KX, NZ), 1, 2).astype(jnp.bfloat16)


def kernel(x_QT, x_SLI, eta, W_QT, b_QT, W_SLI, b_SLI):
    NY, NX = eta.shape
    G = NY // R
    wb = _stack_weights(jnp.stack([W_QT, W_SLI]), jnp.stack([b_QT, b_SLI]))
    return pl.pallas_call(
        _moe_kernel,
        grid=(G,),
        in_specs=[
            pl.BlockSpec((R, NX), lambda i: (i, 0)),
            pl.BlockSpec((NZ, R, NX), lambda i: (0, i, 0)),
            pl.BlockSpec((NZ, R, NX), lambda i: (0, i, 0)),
            pl.BlockSpec((2, NZ, KX), lambda i: (0, 0, 0)),
        ],
        out_specs=pl.BlockSpec((2, NZ, R, NX), lambda i: (0, 0, i, 0)),
        out_shape=jax.ShapeDtypeStruct((2, NZ, NY, NX), jnp.float32),
        compiler_params=pltpu.CompilerParams(
            dimension_semantics=("parallel",), vmem_limit_bytes=100*1024*1024),
    )(eta, x_QT, x_SLI, wb)
